# baseline (speedup 1.0000x reference)
import functools

import jax
import jax.numpy as jnp
from jax import lax
from jax.experimental import pallas as pl
from jax.experimental.pallas import tpu as pltpu
from jax.experimental.pallas import tpu_sc as plsc

_B = 8
_N = 4096
_S = 1024
_K = 32
_C = 256


def _fps_body(x_ref, y_ref, z_ref, idx_ref, cx_ref, cy_ref, cz_ref,
              d_ref, far_ref, bi_ref, bx_ref, by_ref, bz_ref):
    X = x_ref[...]
    Y = y_ref[...]
    Z = z_ref[...]
    d_ref[...] = jnp.full((_B, _N), 1e10, jnp.float32)
    far_ref[...] = jnp.zeros((_B, 128), jnp.int32)

    def step(i, _):
        lane = jax.lax.broadcasted_iota(jnp.int32, (_B, _N), 1)
        lane128 = jax.lax.broadcasted_iota(jnp.int32, (_B, 128), 1)
        zero = jnp.zeros((_B, _N), jnp.float32)
        far = far_ref[:, :1]
        oh = lane == far
        cx = jnp.sum(jnp.where(oh, X, zero), axis=1, keepdims=True)
        cy = jnp.sum(jnp.where(oh, Y, zero), axis=1, keepdims=True)
        cz = jnp.sum(jnp.where(oh, Z, zero), axis=1, keepdims=True)
        sel = lane128 == (i % 128)
        bi_ref[...] = jnp.where(sel, jnp.broadcast_to(far, (_B, 128)), bi_ref[...])
        bx_ref[...] = jnp.where(sel, jnp.broadcast_to(cx, (_B, 128)), bx_ref[...])
        by_ref[...] = jnp.where(sel, jnp.broadcast_to(cy, (_B, 128)), by_ref[...])
        bz_ref[...] = jnp.where(sel, jnp.broadcast_to(cz, (_B, 128)), bz_ref[...])
        dx = X - cx
        dy = Y - cy
        dz = Z - cz
        dist = dx * dx + dy * dy + dz * dz
        D = jnp.minimum(d_ref[...], dist)
        d_ref[...] = D
        m = jnp.max(D, axis=1, keepdims=True)
        nfar = jnp.min(jnp.where(D == m, lane, _N), axis=1, keepdims=True)
        far_ref[...] = jnp.broadcast_to(nfar, (_B, 128))
        return 0

    def block(j, _):
        jax.lax.fori_loop(j * 128, j * 128 + 128, step, 0)
        off = pl.multiple_of(j * 128, 128)
        idx_ref[:, pl.ds(off, 128)] = bi_ref[...]
        cx_ref[:, pl.ds(off, 128)] = bx_ref[...]
        cy_ref[:, pl.ds(off, 128)] = by_ref[...]
        cz_ref[:, pl.ds(off, 128)] = bz_ref[...]
        return 0

    jax.lax.fori_loop(0, _S // 128, block, 0)


def _run_fps(xyz):
    x = xyz[:, :, 0]
    y = xyz[:, :, 1]
    z = xyz[:, :, 2]
    out_shapes = (
        jax.ShapeDtypeStruct((_B, _S), jnp.int32),
        jax.ShapeDtypeStruct((_B, _S), jnp.float32),
        jax.ShapeDtypeStruct((_B, _S), jnp.float32),
        jax.ShapeDtypeStruct((_B, _S), jnp.float32),
    )
    fps_idx, cx, cy, cz = pl.pallas_call(
        _fps_body,
        out_shape=out_shapes,
        scratch_shapes=[
            pltpu.VMEM((_B, _N), jnp.float32),
            pltpu.VMEM((_B, 128), jnp.int32),
            pltpu.VMEM((_B, 128), jnp.int32),
            pltpu.VMEM((_B, 128), jnp.float32),
            pltpu.VMEM((_B, 128), jnp.float32),
            pltpu.VMEM((_B, 128), jnp.float32),
        ],
    )(x, y, z)
    new_xyz = jnp.stack([cx, cy, cz], axis=-1)
    return fps_idx, new_xyz


_INF = 3.4e38
_QW = 256
_QCH = 8


def _dist_body(q_ref, pt_ref, out_ref):
    Q = q_ref[...]
    PT = pt_ref[...]
    mm = jnp.dot(Q, PT, preferred_element_type=jnp.float32)
    qn = (Q[:, 0:1] * Q[:, 0:1] + Q[:, 1:2] * Q[:, 1:2]) + Q[:, 2:3] * Q[:, 2:3]
    pn = (PT[0:1, :] * PT[0:1, :] + PT[1:2, :] * PT[1:2, :]) + PT[2:3, :] * PT[2:3, :]
    out_ref[...] = (mm * (-2.0) + qn) + pn


def _run_dist(new_xyz, xyz):
    perm = (jnp.arange(_N, dtype=jnp.int32) % 16) * 256 + (
        jnp.arange(_N, dtype=jnp.int32) // 16)
    ptp = jnp.swapaxes(xyz, 1, 2)[:, :, perm]

    def body(q_ref, pt_ref, out_ref):
        _dist_body(q_ref.at[0], pt_ref.at[0], out_ref.at[0])
    f = pl.pallas_call(
        body,
        grid=(_B,),
        in_specs=[
            pl.BlockSpec((1, _S, 3), lambda b: (b, 0, 0)),
            pl.BlockSpec((1, 3, _N), lambda b: (b, 0, 0)),
        ],
        out_specs=pl.BlockSpec((1, _S, _N), lambda b: (b, 0, 0)),
        out_shape=jax.ShapeDtypeStruct((_B, _S, _N), jnp.float32),
    )
    return f(new_xyz, ptp).reshape(_B * _S, _N)


def _sc_knngather_body(dist_hbm, table_hbm, aidx_hbm, out_hbm,
                       idx_v, aidx_v, mv, dbuf0, dbuf1,
                       idxq0, idxq1, idxq2, idxq3,
                       rbuf0, rbuf1, rbuf2, rbuf3,
                       dsem0, dsem1, rsem0, rsem1, rsem2, rsem3):
    wid = lax.axis_index("s") * 2 + lax.axis_index("c")
    b = wid // 4

    def mkidx(h, _):
        iota = lax.iota(jnp.int32, 16)
        idx_v[pl.ds(pl.multiple_of(h * 16, 16), 16)] = wid * _QW + h * 16 + iota
        return 0
    lax.fori_loop(0, _QW // 16, mkidx, 0)
    pltpu.sync_copy(aidx_hbm.at[pl.ds(wid * _QW, _QW)], aidx_v)

    nch = _QW // _QCH

    def fire_dist(c, buf, sem):
        pltpu.async_copy(dist_hbm.at[idx_v.at[pl.ds(c * _QCH, _QCH)]], buf, sem)

    def wait_dist(buf, sem):
        pltpu.make_async_copy(
            dist_hbm.at[idx_v.at[pl.ds(0, _QCH)]], buf, sem).wait()

    def fire_rows(idxq, rbuf, sem):
        pltpu.async_copy(table_hbm.at[idxq.at[pl.ds(0, 40)]], rbuf, sem)

    def wait_rows(idxq, rbuf, sem):
        pltpu.make_async_copy(table_hbm.at[idxq.at[pl.ds(0, 40)]], rbuf, sem).wait()

    def drain(qp, rbuf, sem_idxq):
        idxq, sem = sem_idxq
        wait_rows(idxq, rbuf, sem)

        def rrow(r, __):
            for v in range(_C // 16):
                sl = pl.ds(v * 16, 16)
                rbuf[r, sl] = rbuf[r, sl] - rbuf[32, sl]
            return 0
        lax.fori_loop(0, _K, rrow, 0, unroll=2)
        pltpu.sync_copy(rbuf.at[pl.ds(0, _K)],
                        out_hbm.at[pl.ds((wid * _QW + qp) * _K, _K)])

    def extract_query(buf, i, q):
        iota = lax.iota(jnp.int32, 16)

        def mrow(g, _):
            m = jnp.full((16,), _INF, jnp.float32)
            for t in range(16):
                m = jnp.minimum(
                    m, buf[i, pl.ds(pl.multiple_of(g * 256 + t * 16, 16), 16)])
            mv[pl.ds(pl.multiple_of(g * 16, 16), 16)] = m
            return 0
        lax.fori_loop(0, 16, mrow, 0)

        T = mv[pl.ds(0, 16)]
        for g in range(1, 16):
            T = jnp.minimum(T, mv[pl.ds(g * 16, 16)])

        def extract(k, carry):
            T, iA, iB = carry
            iota = lax.iota(jnp.int32, 16)
            gmin = jnp.min(T)
            gs = jnp.full((16,), gmin, jnp.float32)
            lvec = plsc.all_reduce_ffs(T == gs)
            GV = plsc.load_gather(mv, [iota * 16 + lvec])
            gvec = plsc.all_reduce_ffs(GV == gs)
            JV = plsc.load_gather(
                buf, [jnp.full((16,), i, jnp.int32),
                      (gvec * 16 + iota) * 16 + lvec])
            tvec = plsc.all_reduce_ffs(JV == gs)
            nvec = lvec * 256 + gvec * 16 + tvec + b * _N
            iA = jnp.where(iota == k, nvec, iA)
            iB = jnp.where(iota == (k - 16), nvec, iB)
            inf = jnp.full((16,), _INF, jnp.float32)
            j_s = jnp.max(gvec * 16 + tvec)
            off = pl.multiple_of(j_s * 16, 16)
            row = buf[i, pl.ds(off, 16)]
            buf[i, pl.ds(off, 16)] = jnp.where(iota == lvec, inf, row)
            newm = jnp.min(jnp.where(iota == tvec, inf, JV))
            g_s = jnp.max(gvec)
            moff = pl.multiple_of(g_s * 16, 16)
            mrow2 = mv[pl.ds(moff, 16)]
            mv[pl.ds(moff, 16)] = jnp.where(
                iota == lvec, jnp.full((16,), newm, jnp.float32), mrow2)
            newt = jnp.min(jnp.where(iota == gvec,
                                     jnp.full((16,), newm, jnp.float32), GV))
            T2 = jnp.where(iota == lvec,
                           jnp.full((16,), newt, jnp.float32), T)
            return (T2, iA, iB)

        zi = jnp.zeros((16,), jnp.int32)
        _, iA, iB = lax.fori_loop(0, _K, extract, (T, zi, zi))
        qh16 = pl.multiple_of(q - q % 16, 16)
        av = aidx_v[pl.ds(qh16, 16)]
        aid = jnp.max(jnp.where(iota == q % 16, av, jnp.zeros((16,), jnp.int32)))
        aidv = jnp.full((16,), aid, jnp.int32)
        return iA, iB, aidv

    slots = [(idxq0, rbuf0, rsem0), (idxq1, rbuf1, rsem1),
             (idxq2, rbuf2, rsem2), (idxq3, rbuf3, rsem3)]

    def process_chunk(c, buf):
        for i in range(_QCH):
            q = c * _QCH + i
            idxq, rbuf, rsem = slots[i % 4]
            iA, iB, aidv = extract_query(buf, i, q)

            @pl.when(q - 4 >= 0)
            def _():
                drain(q - 4, rbuf, (idxq, rsem))
            idxq[pl.ds(0, 16)] = iA
            idxq[pl.ds(16, 16)] = iB
            idxq[pl.ds(32, 16)] = aidv
            fire_rows(idxq, rbuf, rsem)

    fire_dist(0, dbuf0, dsem0)

    def loop(c, _):
        even = c % 2 == 0

        @pl.when(c + 1 < nch)
        def _():
            @pl.when(even)
            def _():
                fire_dist(c + 1, dbuf1, dsem1)

            @pl.when(jnp.logical_not(even))
            def _():
                fire_dist(c + 1, dbuf0, dsem0)

        @pl.when(even)
        def _():
            wait_dist(dbuf0, dsem0)
            process_chunk(c, dbuf0)

        @pl.when(jnp.logical_not(even))
        def _():
            wait_dist(dbuf1, dsem1)
            process_chunk(c, dbuf1)
        return 0

    lax.fori_loop(0, nch, loop, 0)
    drain(_QW - 4, rbuf0, (idxq0, rsem0))
    drain(_QW - 3, rbuf1, (idxq1, rsem1))
    drain(_QW - 2, rbuf2, (idxq2, rsem2))
    drain(_QW - 1, rbuf3, (idxq3, rsem3))


def _run_knngather(xyz, new_xyz, points, flat_aidx):
    dist = _run_dist(new_xyz, xyz)
    table = points.reshape(_B * _N, _C)
    mesh = plsc.VectorSubcoreMesh(core_axis_name="c", subcore_axis_name="s")
    f = pl.kernel(
        _sc_knngather_body,
        mesh=mesh,
        compiler_params=pltpu.CompilerParams(needs_layout_passes=False),
        out_type=jax.ShapeDtypeStruct((_B * _S * _K, _C), jnp.float32),
        scratch_types=[
            pltpu.VMEM((_QW,), jnp.int32),
            pltpu.VMEM((_QW,), jnp.int32),
            pltpu.VMEM((256,), jnp.float32),
            pltpu.VMEM((_QCH, _N), jnp.float32),
            pltpu.VMEM((_QCH, _N), jnp.float32),
            pltpu.VMEM((48,), jnp.int32),
            pltpu.VMEM((48,), jnp.int32),
            pltpu.VMEM((48,), jnp.int32),
            pltpu.VMEM((48,), jnp.int32),
            pltpu.VMEM((40, _C), jnp.float32),
            pltpu.VMEM((40, _C), jnp.float32),
            pltpu.VMEM((40, _C), jnp.float32),
            pltpu.VMEM((40, _C), jnp.float32),
            pltpu.SemaphoreType.DMA,
            pltpu.SemaphoreType.DMA,
            pltpu.SemaphoreType.DMA,
            pltpu.SemaphoreType.DMA,
            pltpu.SemaphoreType.DMA,
            pltpu.SemaphoreType.DMA,
        ],
    )
    return f(dist, table, flat_aidx)


def kernel(xyz, points):
    fps_idx, new_xyz = _run_fps(xyz)
    boff = (jnp.arange(_B, dtype=jnp.int32) * _N)
    flat_aidx = (fps_idx + boff[:, None]).reshape(-1)
    a = _run_knngather(xyz, new_xyz, points, flat_aidx)
    return (new_xyz, a.reshape(_B, _S, _K, _C))
